# baseline (speedup 1.0000x reference)
import functools

import jax
import jax.numpy as jnp
from jax import lax
from jax.experimental import pallas as pl
from jax.experimental.pallas import tpu as pltpu
from jax.experimental.pallas import tpu_sc as plsc

D = 256
_OFF_AB = 1536
_OFF_IT = 1536 + 512
_OFF_G1 = 1536 + 512 + 1024
_OFF_G2 = _OFF_G1 + 1024
K_IDX = 12
E_BLK = 2048
C_SC = 8
NW = 32


def _prep_body(e_ref, wsm_ref, bstack_ref, p_ref, idx_ref, w_ref, m_ref):
    e = e_ref[...]
    E = e.shape[0]
    f32 = jnp.float32

    def col(i):
        return e[:, i:i + 1]

    segs = []
    i7 = lax.broadcasted_iota(jnp.int32, (E, 7), 1)
    segs.append((lax.shift_right_logical(jnp.broadcast_to(col(11), (E, 7)), i7) & 1).astype(f32))
    i10 = lax.broadcasted_iota(jnp.int32, (E, 10), 1)
    segs.append((lax.shift_right_logical(jnp.broadcast_to(col(12), (E, 10)), i10) & 1).astype(f32))
    i4 = lax.broadcasted_iota(jnp.int32, (E, 4), 1)
    for j in range(9):
        segs.append((lax.shift_right_logical(jnp.broadcast_to(col(29 + j), (E, 4)), i4) & 1).astype(f32))
    i9 = lax.broadcasted_iota(jnp.int32, (1, 9), 1)
    rescale = jnp.where(i9 == 0, 1.0 / 100, jnp.where(i9 == 1, 1.0 / 1023, 0.5))
    segs.append(e[:, 11:20].astype(f32) * rescale)
    for ci, sz in ((3, 3), (4, 7), (5, 16), (6, 2), (7, 8), (8, 4), (9, 2), (10, 2)):
        ii = lax.broadcasted_iota(jnp.int32, (E, sz), 1)
        segs.append((ii == col(ci)).astype(f32))
    for b in range(7):
        ii = lax.broadcasted_iota(jnp.int32, (E, 13), 1)
        segs.append((ii == (col(13 + b) + 6)).astype(f32))
    F = jnp.concatenate(segs, axis=-1)
    F = jnp.concatenate([F, jnp.zeros((E, 256 - 197), f32)], axis=-1)

    bsum = jnp.sum(bstack_ref[...], axis=0, keepdims=True)
    P = jnp.dot(F, wsm_ref[...], preferred_element_type=f32) + bsum

    sp = col(0)
    maskv = jnp.logical_not(jnp.logical_or(sp == 0, sp == 1))
    mf = maskv.astype(f32)
    p_ref[...] = P * mf
    m_ref[...] = maskv.astype(jnp.int32)

    def clip(x, hi):
        return jnp.clip(x, 0, hi)

    idx_cols = [clip(sp, 1535), _OFF_AB + clip(col(1), 511), _OFF_IT + clip(col(2), 1023)]
    for k in range(4):
        idx_cols.append(_OFF_G1 + clip(col(20 + k), 1023))
    for k in range(4):
        idx_cols.append(_OFF_G2 + clip(col(20 + k), 1023))
    idx_cols.append(jnp.zeros((E, 1), jnp.int32))
    idx_ref[...] = jnp.concatenate(idx_cols, axis=-1)

    pp = e[:, 24:28].astype(f32) * (1.0 / 1023.0)
    w_ref[...] = jnp.concatenate(
        [jnp.broadcast_to(mf, (E, 7)), pp * mf, jnp.zeros((E, 5), f32)], axis=-1)


def _bag_body(n_ent, table_hbm, idxf_hbm, wf_hbm, p_hbm, out_hbm,
              idx_v, w_v, rows_v, acc_v, sem):
    epw = n_ent // NW
    wid = lax.axis_index("s") * 2 + lax.axis_index("c")
    base = wid * epw

    def chunk(t, _):
        e0 = base + t * C_SC
        pltpu.sync_copy(idxf_hbm.at[pl.ds(e0 * K_IDX, C_SC * K_IDX)], idx_v)
        pltpu.sync_copy(wf_hbm.at[pl.ds(e0 * 16, C_SC * 16)], w_v)
        pltpu.async_copy(table_hbm.at[idx_v], rows_v, sem).wait()
        pltpu.sync_copy(p_hbm.at[pl.ds(e0, C_SC)], acc_v)

        def ent(i, _2):
            wvec = w_v[pl.ds(i * 16, 16)]
            for v in range(D // 16):
                a = acc_v[i, pl.ds(v * 16, 16)]
                for j in range(K_IDX):
                    a = a + wvec[j] * rows_v[i * K_IDX + j, pl.ds(v * 16, 16)]
                acc_v[i, pl.ds(v * 16, 16)] = a
            return 0

        lax.fori_loop(0, C_SC, ent, 0)
        pltpu.sync_copy(acc_v, out_hbm.at[pl.ds(e0, C_SC)])
        return 0

    lax.fori_loop(0, epw // C_SC, chunk, 0)


def _run_prep(entities, W_small, b_stack):
    N = entities.shape[0]
    f32 = jnp.float32
    grid = (N // E_BLK,)
    return pl.pallas_call(
        _prep_body,
        grid=grid,
        in_specs=[
            pl.BlockSpec((E_BLK, 38), lambda i: (i, 0)),
            pl.BlockSpec((256, D), lambda i: (0, 0)),
            pl.BlockSpec((9, D), lambda i: (0, 0)),
        ],
        out_specs=[
            pl.BlockSpec((E_BLK, D), lambda i: (i, 0)),
            pl.BlockSpec((E_BLK, K_IDX), lambda i: (i, 0)),
            pl.BlockSpec((E_BLK, 16), lambda i: (i, 0)),
            pl.BlockSpec((E_BLK, 1), lambda i: (i, 0)),
        ],
        out_shape=[
            jax.ShapeDtypeStruct((N, D), f32),
            jax.ShapeDtypeStruct((N, K_IDX), jnp.int32),
            jax.ShapeDtypeStruct((N, 16), f32),
            jax.ShapeDtypeStruct((N, 1), jnp.int32),
        ],
    )(entities, W_small, b_stack)


def _run_bag(table, idx_flat, w_flat, P):
    N = P.shape[0]
    mesh = plsc.VectorSubcoreMesh(core_axis_name="c", subcore_axis_name="s")
    bag = functools.partial(
        pl.kernel,
        mesh=mesh,
        out_type=jax.ShapeDtypeStruct((N, D), jnp.float32),
        scratch_types=[
            pltpu.VMEM((C_SC * K_IDX,), jnp.int32),
            pltpu.VMEM((C_SC * 16,), jnp.float32),
            pltpu.VMEM((C_SC * K_IDX, D), jnp.float32),
            pltpu.VMEM((C_SC, D), jnp.float32),
            pltpu.SemaphoreType.DMA,
        ],
    )(functools.partial(_bag_body, N))
    return bag(table, idx_flat, w_flat, P)


def kernel(entities, W_species, b_species, W_ability, b_ability, W_item, b_item,
           W_moveset, b_moveset, W_level, b_level, W_hp, b_hp, W_vol, b_vol,
           W_feat, b_feat, W_onehot, b_onehot):
    N = entities.shape[0]
    W_small = jnp.concatenate(
        [W_level, W_hp, W_vol, W_feat, W_onehot,
         jnp.zeros((256 - 197, D), jnp.float32)], axis=0)
    b_stack = jnp.stack([b_species, b_ability, b_item, b_moveset, b_level,
                         b_hp, b_vol, b_feat, b_onehot], axis=0)
    table = jnp.concatenate([W_species, W_ability, W_item, W_moveset], axis=0)

    P, idx, w, m = _run_prep(entities, W_small, b_stack)
    emb = _run_bag(table, idx.reshape(-1), w.reshape(-1), P)
    mask = m.reshape(N) != 0
    return emb, mask


# speedup vs baseline: 1.5495x; 1.5495x over previous
import functools

import jax
import jax.numpy as jnp
from jax import lax
from jax.experimental import pallas as pl
from jax.experimental.pallas import tpu as pltpu
from jax.experimental.pallas import tpu_sc as plsc

D = 256
_OFF_AB = 1536
_OFF_IT = 1536 + 512
_OFF_G1 = 1536 + 512 + 1024
_OFF_G2 = _OFF_G1 + 1024
K_IDX = 11
E_BLK = 2048
C_SC = 8
NW = 32
NBUF = 4


def _prep_body(e_ref, wsm_ref, bstack_ref, p_ref, idx_ref, w_ref, m_ref):
    e = e_ref[...]
    E = e.shape[0]
    f32 = jnp.float32

    def col(i):
        return e[:, i:i + 1]

    segs = []
    i7 = lax.broadcasted_iota(jnp.int32, (E, 7), 1)
    segs.append((lax.shift_right_logical(jnp.broadcast_to(col(11), (E, 7)), i7) & 1).astype(f32))
    i10 = lax.broadcasted_iota(jnp.int32, (E, 10), 1)
    segs.append((lax.shift_right_logical(jnp.broadcast_to(col(12), (E, 10)), i10) & 1).astype(f32))
    i4 = lax.broadcasted_iota(jnp.int32, (E, 4), 1)
    for j in range(9):
        segs.append((lax.shift_right_logical(jnp.broadcast_to(col(29 + j), (E, 4)), i4) & 1).astype(f32))
    i9 = lax.broadcasted_iota(jnp.int32, (1, 9), 1)
    rescale = jnp.where(i9 == 0, 1.0 / 100, jnp.where(i9 == 1, 1.0 / 1023, 0.5))
    segs.append(e[:, 11:20].astype(f32) * rescale)
    for ci, sz in ((3, 3), (4, 7), (5, 16), (6, 2), (7, 8), (8, 4), (9, 2), (10, 2)):
        ii = lax.broadcasted_iota(jnp.int32, (E, sz), 1)
        segs.append((ii == col(ci)).astype(f32))
    for b in range(7):
        ii = lax.broadcasted_iota(jnp.int32, (E, 13), 1)
        segs.append((ii == (col(13 + b) + 6)).astype(f32))
    F = jnp.concatenate(segs, axis=-1)
    F = jnp.concatenate([F, jnp.zeros((E, 256 - 197), f32)], axis=-1)

    bsum = jnp.sum(bstack_ref[...], axis=0, keepdims=True)
    P = jnp.dot(F, wsm_ref[...], preferred_element_type=f32) + bsum

    sp = col(0)
    maskv = jnp.logical_not(jnp.logical_or(sp == 0, sp == 1))
    mf = maskv.astype(f32)
    p_ref[...] = P * mf
    m_ref[...] = maskv.astype(jnp.int32)

    def clip(x, hi):
        return jnp.clip(x, 0, hi)

    idx_cols = [clip(sp, 1535), _OFF_AB + clip(col(1), 511), _OFF_IT + clip(col(2), 1023)]
    for k in range(4):
        idx_cols.append(_OFF_G1 + clip(col(20 + k), 1023))
    for k in range(4):
        idx_cols.append(_OFF_G2 + clip(col(20 + k), 1023))
    idx_ref[...] = jnp.concatenate(idx_cols, axis=-1)

    pp = e[:, 24:28].astype(f32) * (1.0 / 1023.0)
    w_ref[...] = jnp.concatenate(
        [jnp.broadcast_to(mf, (E, 7)), pp * mf, jnp.zeros((E, 5), f32)], axis=-1)


def _bag_body(n_ent, table_hbm, idxr_hbm, wr_hbm, p_hbm, out_hbm,
              idx_all, w_all, rows_b, p_b, acc_b, gsem, psem, osem):
    epw = n_ent // NW
    n_chunks = epw // C_SC
    n_steps = n_chunks // NBUF
    wid = lax.axis_index("s") * 2 + lax.axis_index("c")
    base = wid * epw

    pltpu.sync_copy(idxr_hbm.at[wid], idx_all)
    pltpu.sync_copy(wr_hbm.at[wid], w_all)

    def fire(t, b):
        e0 = base + t * C_SC
        pltpu.async_copy(table_hbm.at[idx_all.at[t]], rows_b[b], gsem[b])
        pltpu.async_copy(p_hbm.at[pl.ds(e0, C_SC)], p_b[b], psem[b])

    for b in range(NBUF):
        fire(b, b)

    def step(s, _):
        for b in range(NBUF):
            t = s * NBUF + b
            e0 = base + t * C_SC
            pltpu.make_async_copy(table_hbm.at[idx_all.at[t]], rows_b[b], gsem[b]).wait()
            pltpu.make_async_copy(p_hbm.at[pl.ds(e0, C_SC)], p_b[b], psem[b]).wait()

            @pl.when(s > 0)
            def _wait_store():
                pltpu.make_async_copy(acc_b[b], out_hbm.at[pl.ds(e0, C_SC)], osem[b]).wait()

            def ent(i, _2):
                wvec = w_all[t, pl.ds(i * 16, 16)]
                for v in range(D // 16):
                    a = p_b[b][i, pl.ds(v * 16, 16)]
                    for j in range(K_IDX):
                        a = a + wvec[j] * rows_b[b][i * K_IDX + j, pl.ds(v * 16, 16)]
                    acc_b[b][i, pl.ds(v * 16, 16)] = a
                return 0

            lax.fori_loop(0, C_SC, ent, 0)
            pltpu.async_copy(acc_b[b], out_hbm.at[pl.ds(e0, C_SC)], osem[b])

            @pl.when(s < n_steps - 1)
            def _prefetch():
                fire(t + NBUF, b)
        return 0

    lax.fori_loop(0, n_steps, step, 0)
    for b in range(NBUF):
        e0 = base + ((n_steps - 1) * NBUF + b) * C_SC
        pltpu.make_async_copy(acc_b[b], out_hbm.at[pl.ds(e0, C_SC)], osem[b]).wait()


def _run_prep(entities, W_small, b_stack):
    N = entities.shape[0]
    f32 = jnp.float32
    grid = (N // E_BLK,)
    return pl.pallas_call(
        _prep_body,
        grid=grid,
        in_specs=[
            pl.BlockSpec((E_BLK, 38), lambda i: (i, 0)),
            pl.BlockSpec((256, D), lambda i: (0, 0)),
            pl.BlockSpec((9, D), lambda i: (0, 0)),
        ],
        out_specs=[
            pl.BlockSpec((E_BLK, D), lambda i: (i, 0)),
            pl.BlockSpec((E_BLK, K_IDX), lambda i: (i, 0)),
            pl.BlockSpec((E_BLK, 16), lambda i: (i, 0)),
            pl.BlockSpec((E_BLK, 1), lambda i: (i, 0)),
        ],
        out_shape=[
            jax.ShapeDtypeStruct((N, D), f32),
            jax.ShapeDtypeStruct((N, K_IDX), jnp.int32),
            jax.ShapeDtypeStruct((N, 16), f32),
            jax.ShapeDtypeStruct((N, 1), jnp.int32),
        ],
    )(entities, W_small, b_stack)


def _run_bag(table, idx, w, P):
    N = P.shape[0]
    epw = N // NW
    n_chunks = epw // C_SC
    idx_r = idx.reshape(NW, n_chunks, C_SC * K_IDX)
    w_r = w.reshape(NW, n_chunks, C_SC * 16)
    mesh = plsc.VectorSubcoreMesh(core_axis_name="c", subcore_axis_name="s")
    bag = functools.partial(
        pl.kernel,
        mesh=mesh,
        out_type=jax.ShapeDtypeStruct((N, D), jnp.float32),
        scratch_types=[
            pltpu.VMEM((n_chunks, C_SC * K_IDX), jnp.int32),
            pltpu.VMEM((n_chunks, C_SC * 16), jnp.float32),
            [pltpu.VMEM((C_SC * K_IDX, D), jnp.float32)] * NBUF,
            [pltpu.VMEM((C_SC, D), jnp.float32)] * NBUF,
            [pltpu.VMEM((C_SC, D), jnp.float32)] * NBUF,
            [pltpu.SemaphoreType.DMA] * NBUF,
            [pltpu.SemaphoreType.DMA] * NBUF,
            [pltpu.SemaphoreType.DMA] * NBUF,
        ],
    )(functools.partial(_bag_body, N))
    return bag(table, idx_r, w_r, P)


def kernel(entities, W_species, b_species, W_ability, b_ability, W_item, b_item,
           W_moveset, b_moveset, W_level, b_level, W_hp, b_hp, W_vol, b_vol,
           W_feat, b_feat, W_onehot, b_onehot):
    N = entities.shape[0]
    W_small = jnp.concatenate(
        [W_level, W_hp, W_vol, W_feat, W_onehot,
         jnp.zeros((256 - 197, D), jnp.float32)], axis=0)
    b_stack = jnp.stack([b_species, b_ability, b_item, b_moveset, b_level,
                         b_hp, b_vol, b_feat, b_onehot], axis=0)
    table = jnp.concatenate([W_species, W_ability, W_item, W_moveset], axis=0)

    P, idx, w, m = _run_prep(entities, W_small, b_stack)
    emb = _run_bag(table, idx, w, P)
    mask = m.reshape(N) != 0
    return emb, mask


# speedup vs baseline: 3.5812x; 2.3112x over previous
import functools

import jax
import jax.numpy as jnp
from jax import lax
from jax.experimental import pallas as pl
from jax.experimental.pallas import tpu as pltpu
from jax.experimental.pallas import tpu_sc as plsc

D = 256
_OFF_AB = 1536
_OFF_IT = 1536 + 512
_OFF_G1 = 1536 + 512 + 1024
_OFF_G2 = _OFF_G1 + 1024
K_IDX = 11
E_BLK = 2048
C_SC = 8
NW = 32
NBUF = 4


def _prep_body(e_ref, wsm_ref, bstack_ref, p_ref, idx_ref, w_ref, m_ref):
    e = e_ref[...]
    E = e.shape[0]
    f32 = jnp.float32

    def col(i):
        return e[:, i:i + 1]

    segs = []
    i7 = lax.broadcasted_iota(jnp.int32, (E, 7), 1)
    segs.append((lax.shift_right_logical(jnp.broadcast_to(col(11), (E, 7)), i7) & 1).astype(f32))
    i10 = lax.broadcasted_iota(jnp.int32, (E, 10), 1)
    segs.append((lax.shift_right_logical(jnp.broadcast_to(col(12), (E, 10)), i10) & 1).astype(f32))
    i4 = lax.broadcasted_iota(jnp.int32, (E, 4), 1)
    for j in range(9):
        segs.append((lax.shift_right_logical(jnp.broadcast_to(col(29 + j), (E, 4)), i4) & 1).astype(f32))
    i9 = lax.broadcasted_iota(jnp.int32, (1, 9), 1)
    rescale = jnp.where(i9 == 0, 1.0 / 100, jnp.where(i9 == 1, 1.0 / 1023, 0.5))
    segs.append(e[:, 11:20].astype(f32) * rescale)
    for ci, sz in ((3, 3), (4, 7), (5, 16), (6, 2), (7, 8), (8, 4), (9, 2), (10, 2)):
        ii = lax.broadcasted_iota(jnp.int32, (E, sz), 1)
        segs.append((ii == col(ci)).astype(f32))
    for b in range(7):
        ii = lax.broadcasted_iota(jnp.int32, (E, 13), 1)
        segs.append((ii == (col(13 + b) + 6)).astype(f32))
    F = jnp.concatenate(segs, axis=-1)
    F = jnp.concatenate([F, jnp.zeros((E, 256 - 197), f32)], axis=-1)

    bsum = jnp.sum(bstack_ref[...], axis=0, keepdims=True)
    P = jnp.dot(F, wsm_ref[...], preferred_element_type=f32) + bsum

    sp = col(0)
    maskv = jnp.logical_not(jnp.logical_or(sp == 0, sp == 1))
    mf = maskv.astype(f32)
    p_ref[...] = P * mf
    m_ref[...] = maskv.astype(jnp.int32)

    def clip(x, hi):
        return jnp.clip(x, 0, hi)

    idx_cols = [clip(sp, 1535), _OFF_AB + clip(col(1), 511), _OFF_IT + clip(col(2), 1023)]
    for k in range(4):
        idx_cols.append(_OFF_G1 + clip(col(20 + k), 1023))
    for k in range(4):
        idx_cols.append(_OFF_G2 + clip(col(20 + k), 1023))
    idx_ref[...] = jnp.concatenate(idx_cols, axis=-1)

    pp = e[:, 24:28].astype(f32) * (1.0 / 1023.0)
    w_ab = jnp.where(col(1) <= 511, mf, jnp.where(maskv, jnp.float32(jnp.nan), 0.0))
    w_ref[...] = jnp.concatenate(
        [mf, w_ab, jnp.broadcast_to(mf, (E, 5)), pp * mf, jnp.zeros((E, 5), f32)], axis=-1)


def _bag_body(n_ent, table_hbm, idxr_hbm, wr_hbm, p_hbm, out_hbm,
              tsh, idx_all, w_all, rows_b, p_b, acc_b, gsem, psem, osem):
    epw = n_ent // NW
    n_chunks = epw // C_SC
    n_steps = n_chunks // NBUF
    sid = lax.axis_index("s")
    wid = sid * 2 + lax.axis_index("c")
    base = wid * epw
    t_rows = _OFF_G2 + 1024

    t_slice = t_rows // 16
    pltpu.sync_copy(table_hbm.at[pl.ds(sid * t_slice, t_slice)],
                    tsh.at[pl.ds(sid * t_slice, t_slice)])

    pltpu.sync_copy(idxr_hbm.at[wid], idx_all)
    pltpu.sync_copy(wr_hbm.at[wid], w_all)
    plsc.subcore_barrier()

    def fire(t, b):
        e0 = base + t * C_SC
        pltpu.async_copy(tsh.at[idx_all.at[t]], rows_b[b], gsem[b])
        pltpu.async_copy(p_hbm.at[pl.ds(e0, C_SC)], p_b[b], psem[b])

    for b in range(NBUF):
        fire(b, b)

    def step(s, _):
        for b in range(NBUF):
            t = s * NBUF + b
            e0 = base + t * C_SC
            pltpu.make_async_copy(tsh.at[idx_all.at[t]], rows_b[b], gsem[b]).wait()
            pltpu.make_async_copy(p_hbm.at[pl.ds(e0, C_SC)], p_b[b], psem[b]).wait()

            @pl.when(s > 0)
            def _wait_store():
                pltpu.make_async_copy(acc_b[b], out_hbm.at[pl.ds(e0, C_SC)], osem[b]).wait()

            def ent(i, _2):
                wvec = w_all[t, pl.ds(i * 16, 16)]
                for v in range(D // 32):
                    alo = p_b[b][i, pl.ds(v * 32, 16)]
                    ahi = p_b[b][i, pl.ds(v * 32 + 16, 16)]
                    for j in range(K_IDX):
                        x = rows_b[b][i * K_IDX + j, pl.ds(v * 16, 16)]
                        lo = lax.bitcast_convert_type(lax.shift_left(x, 16), jnp.float32)
                        hi = lax.bitcast_convert_type(jnp.bitwise_and(x, jnp.int32(-65536)), jnp.float32)
                        alo = alo + wvec[j] * lo
                        ahi = ahi + wvec[j] * hi
                    acc_b[b][i, pl.ds(v * 32, 16)] = alo
                    acc_b[b][i, pl.ds(v * 32 + 16, 16)] = ahi
                return 0

            lax.fori_loop(0, C_SC, ent, 0)
            pltpu.async_copy(acc_b[b], out_hbm.at[pl.ds(e0, C_SC)], osem[b])

            @pl.when(s < n_steps - 1)
            def _prefetch():
                fire(t + NBUF, b)
        return 0

    lax.fori_loop(0, n_steps, step, 0)
    for b in range(NBUF):
        e0 = base + ((n_steps - 1) * NBUF + b) * C_SC
        pltpu.make_async_copy(acc_b[b], out_hbm.at[pl.ds(e0, C_SC)], osem[b]).wait()


def _run_prep(entities, W_small, b_stack):
    N = entities.shape[0]
    f32 = jnp.float32
    grid = (N // E_BLK,)
    return pl.pallas_call(
        _prep_body,
        grid=grid,
        in_specs=[
            pl.BlockSpec((E_BLK, 38), lambda i: (i, 0)),
            pl.BlockSpec((256, D), lambda i: (0, 0)),
            pl.BlockSpec((9, D), lambda i: (0, 0)),
        ],
        out_specs=[
            pl.BlockSpec((E_BLK, D), lambda i: (i, 0)),
            pl.BlockSpec((E_BLK, K_IDX), lambda i: (i, 0)),
            pl.BlockSpec((E_BLK, 16), lambda i: (i, 0)),
            pl.BlockSpec((E_BLK, 1), lambda i: (i, 0)),
        ],
        out_shape=[
            jax.ShapeDtypeStruct((N, D), f32),
            jax.ShapeDtypeStruct((N, K_IDX), jnp.int32),
            jax.ShapeDtypeStruct((N, 16), f32),
            jax.ShapeDtypeStruct((N, 1), jnp.int32),
        ],
    )(entities, W_small, b_stack)


def _run_bag(table, idx, w, P):
    N = P.shape[0]
    epw = N // NW
    n_chunks = epw // C_SC
    idx_r = idx.reshape(NW, n_chunks, C_SC * K_IDX)
    w_r = w.reshape(NW, n_chunks, C_SC * 16)
    mesh = plsc.VectorSubcoreMesh(core_axis_name="c", subcore_axis_name="s")
    bag = functools.partial(
        pl.kernel,
        mesh=mesh,
        out_type=jax.ShapeDtypeStruct((N, D), jnp.float32),
        scratch_types=[
            pltpu.VMEM_SHARED((_OFF_G2 + 1024, D // 2), jnp.int32),
            pltpu.VMEM((n_chunks, C_SC * K_IDX), jnp.int32),
            pltpu.VMEM((n_chunks, C_SC * 16), jnp.float32),
            [pltpu.VMEM((C_SC * K_IDX, D // 2), jnp.int32)] * NBUF,
            [pltpu.VMEM((C_SC, D), jnp.float32)] * NBUF,
            [pltpu.VMEM((C_SC, D), jnp.float32)] * NBUF,
            [pltpu.SemaphoreType.DMA] * NBUF,
            [pltpu.SemaphoreType.DMA] * NBUF,
            [pltpu.SemaphoreType.DMA] * NBUF,
        ],
    )(functools.partial(_bag_body, N))
    return bag(table, idx_r, w_r, P)


def kernel(entities, W_species, b_species, W_ability, b_ability, W_item, b_item,
           W_moveset, b_moveset, W_level, b_level, W_hp, b_hp, W_vol, b_vol,
           W_feat, b_feat, W_onehot, b_onehot):
    N = entities.shape[0]
    W_small = jnp.concatenate(
        [W_level, W_hp, W_vol, W_feat, W_onehot,
         jnp.zeros((256 - 197, D), jnp.float32)], axis=0)
    b_stack = jnp.stack([b_species, b_ability, b_item, b_moveset, b_level,
                         b_hp, b_vol, b_feat, b_onehot], axis=0)
    table = jnp.concatenate([W_species, W_ability, W_item, W_moveset], axis=0)
    tb = table.astype(jnp.bfloat16).reshape(-1, 8, 2, 16).transpose(0, 1, 3, 2)
    table = lax.bitcast_convert_type(tb, jnp.int32).reshape(-1, D // 2)

    P, idx, w, m = _run_prep(entities, W_small, b_stack)
    emb = _run_bag(table, idx, w, P)
    mask = m.reshape(N) != 0
    return emb, mask


# speedup vs baseline: 5.5228x; 1.5422x over previous
import functools

import jax
import jax.numpy as jnp
import numpy as np
from jax import lax
from jax.experimental import pallas as pl
from jax.experimental.pallas import tpu as pltpu
from jax.experimental.pallas import tpu_sc as plsc

D = 256
_OFF_AB = 1536
_OFF_IT = 1536 + 512
_OFF_G1 = 1536 + 512 + 1024
_OFF_G2 = _OFF_G1 + 1024
K_IDX = 11
E_BLK = 2048
C_SC = 8
NW = 32
NBUF = 4


def _prep_body(e_ref, wsm_ref, bstack_ref, sel_ref, ci_ref, sh_ref, resc_ref,
               p_ref, idx_ref, w_ref, m_ref):
    e = e_ref[...]
    E = e.shape[0]
    f32 = jnp.float32

    def col(i):
        return e[:, i:i + 1]

    V = jnp.dot(e.astype(f32), sel_ref[...], preferred_element_type=f32)
    VI = V.astype(jnp.int32)
    oh = (VI == ci_ref[...]).astype(f32)
    bits = (lax.shift_right_logical(VI, jnp.broadcast_to(sh_ref[...], (E, 256))) & 1).astype(f32)
    F = oh + bits + V * resc_ref[...]

    bsum = jnp.sum(bstack_ref[...], axis=0, keepdims=True)
    P = jnp.dot(F, wsm_ref[...], preferred_element_type=f32) + bsum

    sp = col(0)
    maskv = jnp.logical_not(jnp.logical_or(sp == 0, sp == 1))
    mf = maskv.astype(f32)
    p_ref[...] = P * mf
    m_ref[...] = maskv.astype(jnp.int32)

    def clip(x, hi):
        return jnp.clip(x, 0, hi)

    idx_cols = [clip(sp, 1535), _OFF_AB + clip(col(1), 511), _OFF_IT + clip(col(2), 1023)]
    for k in range(4):
        idx_cols.append(_OFF_G1 + clip(col(20 + k), 1023))
    for k in range(4):
        idx_cols.append(_OFF_G2 + clip(col(20 + k), 1023))
    idx_ref[...] = jnp.concatenate(idx_cols, axis=-1)

    pp = e[:, 24:28].astype(f32) * (1.0 / 1023.0)
    w_ab = jnp.where(col(1) <= 511, mf, jnp.where(maskv, jnp.float32(jnp.nan), 0.0))
    w_ref[...] = jnp.concatenate(
        [mf, w_ab, jnp.broadcast_to(mf, (E, 5)), pp * mf, jnp.zeros((E, 5), f32)], axis=-1)


def _bag_body(n_ent, table_hbm, idxr_hbm, wr_hbm, p_hbm, out_hbm,
              tsh, idx_all, w_all, rows_b, p_b, acc_b, gsem, psem, osem):
    epw = n_ent // NW
    n_chunks = epw // C_SC
    n_steps = n_chunks // NBUF
    sid = lax.axis_index("s")
    wid = sid * 2 + lax.axis_index("c")
    base = wid * epw
    t_rows = _OFF_G2 + 1024

    t_slice = t_rows // 16
    pltpu.sync_copy(table_hbm.at[pl.ds(sid * t_slice, t_slice)],
                    tsh.at[pl.ds(sid * t_slice, t_slice)])

    pltpu.sync_copy(idxr_hbm.at[wid], idx_all)
    pltpu.sync_copy(wr_hbm.at[wid], w_all)
    plsc.subcore_barrier()

    def fire(t, b):
        e0 = base + t * C_SC
        pltpu.async_copy(tsh.at[idx_all.at[t]], rows_b[b], gsem[b])
        pltpu.async_copy(p_hbm.at[pl.ds(e0, C_SC)], p_b[b], psem[b])

    for b in range(NBUF):
        fire(b, b)

    def step(s, _):
        for b in range(NBUF):
            t = s * NBUF + b
            e0 = base + t * C_SC
            pltpu.make_async_copy(tsh.at[idx_all.at[t]], rows_b[b], gsem[b]).wait()
            pltpu.make_async_copy(p_hbm.at[pl.ds(e0, C_SC)], p_b[b], psem[b]).wait()

            @pl.when(s > 0)
            def _wait_store():
                pltpu.make_async_copy(acc_b[b], out_hbm.at[pl.ds(e0, C_SC)], osem[b]).wait()

            def ent(i, _2):
                wvec = w_all[t, pl.ds(i * 16, 16)]
                for v in range(D // 32):
                    alo = p_b[b][i, pl.ds(v * 32, 16)]
                    ahi = p_b[b][i, pl.ds(v * 32 + 16, 16)]
                    for j in range(K_IDX):
                        x = rows_b[b][i * K_IDX + j, pl.ds(v * 16, 16)]
                        lo = lax.bitcast_convert_type(lax.shift_left(x, 16), jnp.float32)
                        hi = lax.bitcast_convert_type(jnp.bitwise_and(x, jnp.int32(-65536)), jnp.float32)
                        alo = alo + wvec[j] * lo
                        ahi = ahi + wvec[j] * hi
                    acc_b[b][i, pl.ds(v * 32, 16)] = alo
                    acc_b[b][i, pl.ds(v * 32 + 16, 16)] = ahi
                return 0

            lax.fori_loop(0, C_SC, ent, 0)
            pltpu.async_copy(acc_b[b], out_hbm.at[pl.ds(e0, C_SC)], osem[b])

            @pl.when(s < n_steps - 1)
            def _prefetch():
                fire(t + NBUF, b)
        return 0

    lax.fori_loop(0, n_steps, step, 0)
    for b in range(NBUF):
        e0 = base + ((n_steps - 1) * NBUF + b) * C_SC
        pltpu.make_async_copy(acc_b[b], out_hbm.at[pl.ds(e0, C_SC)], osem[b]).wait()


def _prep_consts():
    S = np.zeros((38, 256), np.float32)
    CI = np.full((1, 256), -1000000, np.int32)
    SH = np.full((1, 256), 31, np.int32)
    RESC = np.zeros((1, 256), np.float32)
    c = 0
    for b in range(7):
        S[11, c] = 1; SH[0, c] = b; c += 1
    for b in range(10):
        S[12, c] = 1; SH[0, c] = b; c += 1
    for j in range(9):
        for b in range(4):
            S[29 + j, c] = 1; SH[0, c] = b; c += 1
    feat_resc = [1.0 / 100, 1.0 / 1023] + [0.5] * 7
    for k in range(9):
        S[11 + k, c] = 1; RESC[0, c] = feat_resc[k]; c += 1
    for f, sz in ((3, 3), (4, 7), (5, 16), (6, 2), (7, 8), (8, 4), (9, 2), (10, 2)):
        for v in range(sz):
            S[f, c] = 1; CI[0, c] = v; c += 1
    for i in range(7):
        for v in range(13):
            S[13 + i, c] = 1; CI[0, c] = v - 6; c += 1
    assert c == 197
    return jnp.asarray(S), jnp.asarray(CI), jnp.asarray(SH), jnp.asarray(RESC)


def _run_prep(entities, W_small, b_stack, sel, ci, sh, resc):
    N = entities.shape[0]
    f32 = jnp.float32
    grid = (N // E_BLK,)
    return pl.pallas_call(
        _prep_body,
        grid=grid,
        in_specs=[
            pl.BlockSpec((E_BLK, 38), lambda i: (i, 0)),
            pl.BlockSpec((256, D), lambda i: (0, 0)),
            pl.BlockSpec((9, D), lambda i: (0, 0)),
            pl.BlockSpec((38, 256), lambda i: (0, 0)),
            pl.BlockSpec((1, 256), lambda i: (0, 0)),
            pl.BlockSpec((1, 256), lambda i: (0, 0)),
            pl.BlockSpec((1, 256), lambda i: (0, 0)),
        ],
        out_specs=[
            pl.BlockSpec((E_BLK, D), lambda i: (i, 0)),
            pl.BlockSpec((E_BLK, K_IDX), lambda i: (i, 0)),
            pl.BlockSpec((E_BLK, 16), lambda i: (i, 0)),
            pl.BlockSpec((E_BLK, 1), lambda i: (i, 0)),
        ],
        out_shape=[
            jax.ShapeDtypeStruct((N, D), f32),
            jax.ShapeDtypeStruct((N, K_IDX), jnp.int32),
            jax.ShapeDtypeStruct((N, 16), f32),
            jax.ShapeDtypeStruct((N, 1), jnp.int32),
        ],
    )(entities, W_small, b_stack, sel, ci, sh, resc)


def _run_bag(table, idx, w, P):
    N = P.shape[0]
    epw = N // NW
    n_chunks = epw // C_SC
    idx_r = idx.reshape(NW, n_chunks, C_SC * K_IDX)
    w_r = w.reshape(NW, n_chunks, C_SC * 16)
    mesh = plsc.VectorSubcoreMesh(core_axis_name="c", subcore_axis_name="s")
    bag = functools.partial(
        pl.kernel,
        mesh=mesh,
        out_type=jax.ShapeDtypeStruct((N, D), jnp.float32),
        scratch_types=[
            pltpu.VMEM_SHARED((_OFF_G2 + 1024, D // 2), jnp.int32),
            pltpu.VMEM((n_chunks, C_SC * K_IDX), jnp.int32),
            pltpu.VMEM((n_chunks, C_SC * 16), jnp.float32),
            [pltpu.VMEM((C_SC * K_IDX, D // 2), jnp.int32)] * NBUF,
            [pltpu.VMEM((C_SC, D), jnp.float32)] * NBUF,
            [pltpu.VMEM((C_SC, D), jnp.float32)] * NBUF,
            [pltpu.SemaphoreType.DMA] * NBUF,
            [pltpu.SemaphoreType.DMA] * NBUF,
            [pltpu.SemaphoreType.DMA] * NBUF,
        ],
    )(functools.partial(_bag_body, N))
    return bag(table, idx_r, w_r, P)


def kernel(entities, W_species, b_species, W_ability, b_ability, W_item, b_item,
           W_moveset, b_moveset, W_level, b_level, W_hp, b_hp, W_vol, b_vol,
           W_feat, b_feat, W_onehot, b_onehot):
    N = entities.shape[0]
    W_small = jnp.concatenate(
        [W_level, W_hp, W_vol, W_feat, W_onehot,
         jnp.zeros((256 - 197, D), jnp.float32)], axis=0)
    b_stack = jnp.stack([b_species, b_ability, b_item, b_moveset, b_level,
                         b_hp, b_vol, b_feat, b_onehot], axis=0)
    table = jnp.concatenate([W_species, W_ability, W_item, W_moveset], axis=0)
    tb = table.astype(jnp.bfloat16).reshape(-1, 8, 2, 16).transpose(0, 1, 3, 2)
    table = lax.bitcast_convert_type(tb, jnp.int32).reshape(-1, D // 2)

    sel, ci, sh, resc = _prep_consts()
    P, idx, w, m = _run_prep(entities, W_small, b_stack, sel, ci, sh, resc)
    emb = _run_bag(table, idx, w, P)
    mask = m.reshape(N) != 0
    return emb, mask
